# baseline (speedup 1.0000x reference)
import functools

import jax
import jax.numpy as jnp
from jax import lax
from jax.experimental import pallas as pl
from jax.experimental.pallas import tpu as pltpu
from jax.experimental.pallas import tpu_sc as plsc

_L = 16


@functools.partial(jax.jit, static_argnames=("seq_len", "tok_per_batch"))
def _sc_gather(flat_rows, idx_padded, seq_len, tok_per_batch):
    num_rows, d = flat_rows.shape
    info = plsc.get_sparse_core_info()
    nc, ns = info.num_cores, info.num_subcores
    nw = nc * ns
    n_idx = idx_padded.shape[0] - _L
    b_per_w = n_idx // nw
    n_vec = -(-b_per_w // _L)
    idx_buf = n_vec * _L

    mesh = plsc.VectorSubcoreMesh(core_axis_name="c", subcore_axis_name="s")

    nbuf = 3
    ch = 8
    n_ch = b_per_w // ch

    @functools.partial(
        pl.kernel,
        mesh=mesh,
        out_type=jax.ShapeDtypeStruct((n_idx, d), jnp.float32),
        scratch_types=[
            pltpu.VMEM((idx_buf,), jnp.int32),
            pltpu.VMEM((nbuf, ch, d), jnp.float32),
            pltpu.SemaphoreType.DMA,
            pltpu.SemaphoreType.DMA,
            pltpu.SemaphoreType.DMA,
            pltpu.SemaphoreType.DMA,
            pltpu.SemaphoreType.DMA,
            pltpu.SemaphoreType.DMA,
        ],
    )
    def k(table_hbm, idx_hbm, out_hbm, idx_v, bufs, g0, g1, g2, w0, w1, w2):
        gs, ws = [g0, g1, g2], [w0, w1, w2]
        wid = lax.axis_index("s") * nc + lax.axis_index("c")
        base = wid * b_per_w
        offset = (base // tok_per_batch) * jnp.int32(seq_len)
        pltpu.sync_copy(idx_hbm.at[pl.ds(base, idx_buf)], idx_v)
        for t in range(n_vec):
            sl = pl.ds(t * _L, _L)
            idx_v[sl] = idx_v[sl] + offset

        g_d, w_d = {}, {}

        def start_gather(c):
            b = c % nbuf
            g_d[c] = pltpu.async_copy(
                table_hbm.at[idx_v.at[pl.ds(c * ch, ch)]], bufs.at[b], gs[b]
            )

        def start_write(c):
            b = c % nbuf
            w_d[c] = pltpu.async_copy(
                bufs.at[b], out_hbm.at[pl.ds(base + c * ch, ch)], ws[b]
            )

        for c in range(min(nbuf, n_ch)):
            start_gather(c)
        for c in range(n_ch):
            g_d[c].wait()
            start_write(c)
            if c + nbuf < n_ch:
                w_d[c].wait()
                start_gather(c + nbuf)
        for c in range(max(0, n_ch - nbuf), n_ch):
            w_d[c].wait()

    return k(flat_rows, idx_padded)


def kernel(inputs, positions):
    mb, s, d = inputs.shape
    _, t = positions.shape
    flat_rows = inputs.reshape(mb * s, d)
    pos_flat = positions.reshape(mb * t)
    pos_padded = jnp.concatenate(
        [pos_flat, jnp.zeros((_L,), jnp.int32)]
    )
    out = _sc_gather(flat_rows, pos_padded, s, t)
    return out.reshape(mb, t, d)


# speedup vs baseline: 1.0286x; 1.0286x over previous
import functools

import jax
import jax.numpy as jnp
from jax import lax
from jax.experimental import pallas as pl
from jax.experimental.pallas import tpu as pltpu
from jax.experimental.pallas import tpu_sc as plsc

_L = 16


@functools.partial(jax.jit, static_argnames=("seq_len", "tok_per_batch"))
def _sc_gather(flat_rows, idx_flat, seq_len, tok_per_batch):
    num_rows, d = flat_rows.shape
    n_idx = idx_flat.shape[0]
    info = plsc.get_sparse_core_info()
    nc, ns = info.num_cores, info.num_subcores
    nw = nc * ns
    b_per_w = n_idx // nw
    n_full = b_per_w // _L
    tail = b_per_w - n_full * _L

    mesh = plsc.VectorSubcoreMesh(core_axis_name="c", subcore_axis_name="s")

    @functools.partial(
        pl.kernel,
        mesh=mesh,
        out_type=jax.ShapeDtypeStruct((n_idx, d), jnp.float32),
        scratch_types=[
            pltpu.VMEM((b_per_w,), jnp.int32),
            pltpu.VMEM((b_per_w, d), jnp.float32),
            pltpu.SemaphoreType.DMA,
        ],
    )
    def k(table_hbm, idx_hbm, out_hbm, idx_v, rows_v, sem):
        wid = lax.axis_index("s") * nc + lax.axis_index("c")
        base = wid * b_per_w
        offset = (base // tok_per_batch) * jnp.int32(seq_len)
        pltpu.sync_copy(idx_hbm.at[pl.ds(base, b_per_w)], idx_v)
        for t in range(n_full):
            sl = pl.ds(t * _L, _L)
            idx_v[sl] = idx_v[sl] + offset
        if tail:
            sl = pl.ds(b_per_w - _L, _L)
            lane = lax.iota(jnp.int32, _L)
            idx_v[sl] = idx_v[sl] + jnp.where(lane >= _L - tail, offset, 0)
        pltpu.async_copy(table_hbm.at[idx_v], rows_v, sem).wait()
        pltpu.sync_copy(rows_v, out_hbm.at[pl.ds(base, b_per_w)])

    return k(flat_rows, idx_flat)


def kernel(inputs, positions):
    mb, s, d = inputs.shape
    _, t = positions.shape
    flat_rows = inputs.reshape(mb * s, d)
    pos_flat = positions.reshape(mb * t)
    out = _sc_gather(flat_rows, pos_flat, s, t)
    return out.reshape(mb, t, d)
